# baseline (speedup 1.0000x reference)
import jax
import jax.numpy as jnp
from jax import lax
from jax.experimental import pallas as pl
from jax.experimental.pallas import tpu as pltpu
from jax.experimental.pallas import tpu_sc as plsc

NC, NS, L = 2, 16, 16
NW = NC * NS
H = 768
HS = H // L
S, B = 2048, 4
ROWS = S * B
RPW = ROWS // NW
POSW = S // NW
CHUNK = 16
NCHUNK = RPW // CHUNK
PPC = CHUNK // B
EPS = 1e-12

_GDN = lax.GatherDimensionNumbers(offset_dims=(), collapsed_slice_dims=(0,),
                                  start_index_map=(0,))


def _gat(vec, idx):
    return lax.gather(vec, jnp.reshape(idx, (L, 1)), _GDN, slice_sizes=(1,),
                      mode=lax.GatherScatterMode.PROMISE_IN_BOUNDS)


def _lane_bcast(vec, lane):
    return _gat(vec, jnp.full((L,), lane, jnp.int32))


def _allsum(v):
    iota = lax.iota(jnp.int32, L)
    for sh in (8, 4, 2, 1):
        v = v + _gat(v, jnp.bitwise_xor(iota, sh))
    return v


def _rsqrt(vv):
    ii = lax.bitcast_convert_type(vv, jnp.int32)
    ii = jnp.int32(0x5F3759DF) - lax.shift_right_logical(ii, 1)
    y = lax.bitcast_convert_type(ii, jnp.float32)
    hv = vv * 0.5
    for _ in range(3):
        y = y * (1.5 - hv * y * y)
    return y


def _body(seq_h, seg_h, tok_h, pos_h, segtab_h, gam_h, bet_h, out_h,
          idx_ref, segi_ref, segf_ref, pos_ref, segtab_ref, sdiff_ref,
          gam_ref, bet_ref, tok0, tok1, ob0, ob1, gsem0, gsem1, osem0,
          osem1):
    wid = lax.axis_index("s") * NC + lax.axis_index("c")
    rbase = wid * RPW

    def gather_start(c, buf, sem):
        pltpu.async_copy(tok_h.at[idx_ref.at[c]], buf, sem)

    pltpu.sync_copy(seq_h.at[pl.ds(wid * NCHUNK, NCHUNK)], idx_ref)
    gather_start(0, tok0, gsem0)
    pltpu.sync_copy(seg_h.at[pl.ds(rbase, RPW)], segi_ref)
    pltpu.sync_copy(pos_h.at[pl.ds(wid * POSW, POSW)], pos_ref)
    pltpu.sync_copy(segtab_h, segtab_ref)
    pltpu.sync_copy(gam_h, gam_ref)
    pltpu.sync_copy(bet_h, bet_ref)

    def sdiff_body(j, _):
        sl = pl.ds(j * L, L)
        sdiff_ref[sl] = segtab_ref[1, sl] - segtab_ref[0, sl]
        return 0
    lax.fori_loop(0, HS, sdiff_body, 0)

    def segf_body(j, _):
        sl = pl.ds(j * L, L)
        segf_ref[sl] = segi_ref[sl].astype(jnp.float32)
        return 0
    lax.fori_loop(0, RPW // L, segf_body, 0)

    zero16 = jnp.zeros((L,), jnp.float32)

    def gather_wait(c, buf, sem):
        pltpu.make_async_copy(tok_h.at[idx_ref.at[c]], buf, sem).wait()

    pbase = wid * POSW

    def out_start(c, buf, sem):
        pltpu.async_copy(buf, out_h.at[pl.ds(pbase + c * PPC, PPC)], sem)

    def out_wait(c, buf, sem):
        pltpu.make_async_copy(
            buf, out_h.at[pl.ds(pbase + c * PPC, PPC)], sem).wait()

    def compute(c, tbuf, obuf):
        grp = segf_ref[pl.ds(c * CHUNK, L)]
        for pp in range(PPC):
            s_loc = c * PPC + pp
            segs = [_lane_bcast(grp, pp * B + k) for k in range(B)]
            rows = [pp * B + k for k in range(B)]

            def p1(j, carry):
                sl = pl.ds(j * L, L)
                bias = pos_ref[s_loc, sl] + segtab_ref[0, sl]
                d = sdiff_ref[sl]
                acc = []
                for k in range(B):
                    x = tbuf[rows[k], sl] + bias + segs[k] * d
                    tbuf[rows[k], sl] = x
                    acc.append(carry[2 * k] + x)
                    acc.append(carry[2 * k + 1] + x * x)
                return tuple(acc)

            with jax.named_scope("PH1"):
                st = plsc.parallel_loop(0, HS, carry=(zero16,) * (2 * B))(p1)
            mbs, rbs = [], []
            with jax.named_scope("PHS"):
              for k in range(B):
                mb = _allsum(st[2 * k]) * (1.0 / H)
                vv = _allsum(st[2 * k + 1]) * (1.0 / H) - mb * mb + EPS
                mbs.append(mb)
                rbs.append(_rsqrt(vv))

            def p2(j):
                sl = pl.ds(j * L, L)
                g = gam_ref[sl]
                bt = bet_ref[sl]
                for k in range(B):
                    x = tbuf[rows[k], sl]
                    obuf[pp, k, sl] = (x - mbs[k]) * (g * rbs[k]) + bt
            with jax.named_scope("PH2"):
                plsc.parallel_loop(0, HS)(p2)

    def half(c, tbuf, obuf, gsem, osem, ntbuf, ngsem):
        @pl.when(c + 1 < NCHUNK)
        def _():
            gather_start(c + 1, ntbuf, ngsem)
        with jax.named_scope("GW"):
            gather_wait(c, tbuf, gsem)
        @pl.when(c >= 2)
        def _():
            out_wait(c - 2, obuf, osem)
        compute(c, tbuf, obuf)
        out_start(c, obuf, osem)

    def pair(p, _):
        c = 2 * p
        half(c, tok0, ob0, gsem0, osem0, tok1, gsem1)
        half(c + 1, tok1, ob1, gsem1, osem1, tok0, gsem0)
        return 0
    lax.fori_loop(0, NCHUNK // 2, pair, 0)

    out_wait(NCHUNK - 2, ob0, osem0)
    out_wait(NCHUNK - 1, ob1, osem1)


@jax.jit
def _emb_ln(seq2d, segf, token_table, position_table, segment_table,
            gamma, beta):
    mesh = plsc.VectorSubcoreMesh(core_axis_name="c", subcore_axis_name="s",
                                  num_cores=NC, num_subcores=NS)
    f = pl.kernel(
        _body,
        out_type=jax.ShapeDtypeStruct((S, B, H), jnp.float32),
        mesh=mesh,
        scratch_types=[
            pltpu.VMEM((NCHUNK, CHUNK), jnp.int32),
            pltpu.VMEM((RPW,), jnp.int32),
            pltpu.VMEM((RPW,), jnp.float32),
            pltpu.VMEM((POSW, H), jnp.float32),
            pltpu.VMEM((2, H), jnp.float32),
            pltpu.VMEM((H,), jnp.float32),
            pltpu.VMEM((H,), jnp.float32),
            pltpu.VMEM((H,), jnp.float32),
            pltpu.VMEM((CHUNK, H), jnp.float32),
            pltpu.VMEM((CHUNK, H), jnp.float32),
            pltpu.VMEM((PPC, B, H), jnp.float32),
            pltpu.VMEM((PPC, B, H), jnp.float32),
            pltpu.SemaphoreType.DMA,
            pltpu.SemaphoreType.DMA,
            pltpu.SemaphoreType.DMA,
            pltpu.SemaphoreType.DMA,
        ],
    )
    return f(seq2d, segf, token_table, position_table, segment_table,
             gamma, beta)


def kernel(seq, seg, token_table, position_table, segment_table, gamma, beta):
    s, b = seq.shape
    seq2d = seq.reshape(ROWS // CHUNK, CHUNK)
    segr = seg.reshape(ROWS)
    return _emb_ln(seq2d, segr, token_table, position_table, segment_table,
                   gamma, beta)


# speedup vs baseline: 1.2748x; 1.2748x over previous
import jax
import jax.numpy as jnp
from jax import lax
from jax.experimental import pallas as pl
from jax.experimental.pallas import tpu as pltpu
from jax.experimental.pallas import tpu_sc as plsc

NC, NS, L = 2, 16, 16
NW = NC * NS
H = 768
HS = H // L
S, B = 2048, 4
ROWS = S * B
RPW = ROWS // NW
POSW = S // NW
CHUNK = 16
NCHUNK = RPW // CHUNK
PPC = CHUNK // B
EPS = 1e-12

_GDN = lax.GatherDimensionNumbers(offset_dims=(), collapsed_slice_dims=(0,),
                                  start_index_map=(0,))


def _gat(vec, idx):
    return lax.gather(vec, jnp.reshape(idx, (L, 1)), _GDN, slice_sizes=(1,),
                      mode=lax.GatherScatterMode.PROMISE_IN_BOUNDS)


def _lane_bcast(vec, lane):
    return _gat(vec, jnp.full((L,), lane, jnp.int32))


def _allsum(v):
    iota = lax.iota(jnp.int32, L)
    for sh in (8, 4, 2, 1):
        v = v + _gat(v, jnp.bitwise_xor(iota, sh))
    return v


def _rsqrt(vv):
    ii = lax.bitcast_convert_type(vv, jnp.int32)
    ii = jnp.int32(0x5F3759DF) - lax.shift_right_logical(ii, 1)
    y = lax.bitcast_convert_type(ii, jnp.float32)
    hv = vv * 0.5
    for _ in range(3):
        y = y * (1.5 - hv * y * y)
    return y


def _body(seq_h, seg_h, tok_h, pos_h, segtab_h, gam_h, bet_h, out_h,
          idx_ref, segi_ref, segf_ref, pos_ref, segtab_ref, sdiff_ref,
          gam_ref, bet_ref, tok0, tok1, ob0, ob1, gsem0, gsem1, osem0,
          osem1):
    wid = lax.axis_index("s") * NC + lax.axis_index("c")
    rbase = wid * RPW

    def gather_start(c, buf, sem):
        pltpu.async_copy(tok_h.at[idx_ref.at[c]], buf, sem)

    pltpu.sync_copy(seq_h.at[pl.ds(wid * NCHUNK, NCHUNK)], idx_ref)
    gather_start(0, tok0, gsem0)
    pltpu.sync_copy(seg_h.at[pl.ds(rbase, RPW)], segi_ref)
    pltpu.sync_copy(pos_h.at[pl.ds(wid * POSW, POSW)], pos_ref)
    pltpu.sync_copy(segtab_h, segtab_ref)
    pltpu.sync_copy(gam_h, gam_ref)
    pltpu.sync_copy(bet_h, bet_ref)

    def sdiff_body(j, _):
        sl = pl.ds(j * L, L)
        sdiff_ref[sl] = segtab_ref[1, sl] - segtab_ref[0, sl]
        return 0
    lax.fori_loop(0, HS, sdiff_body, 0)

    def segf_body(j, _):
        sl = pl.ds(j * L, L)
        segf_ref[sl] = segi_ref[sl].astype(jnp.float32)
        return 0
    lax.fori_loop(0, RPW // L, segf_body, 0)

    zero16 = jnp.zeros((L,), jnp.float32)

    def gather_wait(c, buf, sem):
        pltpu.make_async_copy(tok_h.at[idx_ref.at[c]], buf, sem).wait()

    pbase = wid * POSW

    def out_start(c, buf, sem):
        pltpu.async_copy(buf, out_h.at[pl.ds(pbase + c * PPC, PPC)], sem)

    def out_wait(c, buf, sem):
        pltpu.make_async_copy(
            buf, out_h.at[pl.ds(pbase + c * PPC, PPC)], sem).wait()

    def compute(c, tbuf, obuf):
        grp = segf_ref[pl.ds(c * CHUNK, L)]
        for ph in range(PPC // 2):
            pA, pB = 2 * ph, 2 * ph + 1
            sA, sB = c * PPC + pA, c * PPC + pB
            nr = 2 * B
            segs = [_lane_bcast(grp, pA * B + k) for k in range(nr)]
            rows = [pA * B + k for k in range(nr)]

            def p1(j, carry):
                sl = pl.ds(j * L, L)
                s0 = segtab_ref[0, sl]
                bA = pos_ref[sA, sl] + s0
                bB = pos_ref[sB, sl] + s0
                d = sdiff_ref[sl]
                acc = []
                for k in range(nr):
                    bias = bA if k < B else bB
                    x = tbuf[rows[k], sl] + bias + segs[k] * d
                    tbuf[rows[k], sl] = x
                    acc.append(carry[2 * k] + x)
                    acc.append(carry[2 * k + 1] + x * x)
                return tuple(acc)

            st = plsc.parallel_loop(0, HS, carry=(zero16,) * (2 * nr))(p1)
            mbs, rbs = [], []
            for k in range(nr):
                mb = _allsum(st[2 * k]) * (1.0 / H)
                vv = _allsum(st[2 * k + 1]) * (1.0 / H) - mb * mb + EPS
                mbs.append(mb)
                rbs.append(_rsqrt(vv))

            def p2(j):
                sl = pl.ds(j * L, L)
                g = gam_ref[sl]
                bt = bet_ref[sl]
                for k in range(nr):
                    x = tbuf[rows[k], sl]
                    pp = pA if k < B else pB
                    obuf[pp, k % B, sl] = (x - mbs[k]) * (g * rbs[k]) + bt
            plsc.parallel_loop(0, HS)(p2)

    def half(c, tbuf, obuf, gsem, osem, ntbuf, ngsem):
        @pl.when(c + 1 < NCHUNK)
        def _():
            gather_start(c + 1, ntbuf, ngsem)
        gather_wait(c, tbuf, gsem)
        @pl.when(c >= 2)
        def _():
            out_wait(c - 2, obuf, osem)
        compute(c, tbuf, obuf)
        out_start(c, obuf, osem)

    def pair(p, _):
        c = 2 * p
        half(c, tok0, ob0, gsem0, osem0, tok1, gsem1)
        half(c + 1, tok1, ob1, gsem1, osem1, tok0, gsem0)
        return 0
    lax.fori_loop(0, NCHUNK // 2, pair, 0)

    out_wait(NCHUNK - 2, ob0, osem0)
    out_wait(NCHUNK - 1, ob1, osem1)


@jax.jit
def _emb_ln(seq2d, segf, token_table, position_table, segment_table,
            gamma, beta):
    mesh = plsc.VectorSubcoreMesh(core_axis_name="c", subcore_axis_name="s",
                                  num_cores=NC, num_subcores=NS)
    f = pl.kernel(
        _body,
        out_type=jax.ShapeDtypeStruct((S, B, H), jnp.float32),
        mesh=mesh,
        scratch_types=[
            pltpu.VMEM((NCHUNK, CHUNK), jnp.int32),
            pltpu.VMEM((RPW,), jnp.int32),
            pltpu.VMEM((RPW,), jnp.float32),
            pltpu.VMEM((POSW, H), jnp.float32),
            pltpu.VMEM((2, H), jnp.float32),
            pltpu.VMEM((H,), jnp.float32),
            pltpu.VMEM((H,), jnp.float32),
            pltpu.VMEM((H,), jnp.float32),
            pltpu.VMEM((CHUNK, H), jnp.float32),
            pltpu.VMEM((CHUNK, H), jnp.float32),
            pltpu.VMEM((PPC, B, H), jnp.float32),
            pltpu.VMEM((PPC, B, H), jnp.float32),
            pltpu.SemaphoreType.DMA,
            pltpu.SemaphoreType.DMA,
            pltpu.SemaphoreType.DMA,
            pltpu.SemaphoreType.DMA,
        ],
    )
    return f(seq2d, segf, token_table, position_table, segment_table,
             gamma, beta)


def kernel(seq, seg, token_table, position_table, segment_table, gamma, beta):
    s, b = seq.shape
    seq2d = seq.reshape(ROWS // CHUNK, CHUNK)
    segr = seg.reshape(ROWS)
    return _emb_ln(seq2d, segr, token_table, position_table, segment_table,
                   gamma, beta)
